# baseline (speedup 1.0000x reference)
import functools

import jax
import jax.numpy as jnp
from jax import lax
from jax.experimental import pallas as pl
from jax.experimental.pallas import tpu as pltpu
from jax.experimental.pallas import tpu_sc as plsc

_B, _C, _H, _W = 64, 96, 14, 14
_P = 14
_NPIX = _H * _W
_N = _NPIX + _P
_K = 9
_R = 32
_C2 = 2 * _C
_EPS = 1e-5
_NB = 8
_G = _B // _NB

_ROWS = _B * _N
_NW = 32
_RPW = _ROWS // _NW
_BLK = 6
_NBLK = _RPW // _BLK
_C2P = 256


def _phase1_body(x_ref, w_ref, b_ref, y_ref, st_ref):
    i = pl.program_id(0)
    upd = jnp.zeros((_C, 2), jnp.float32)
    for s in range(_NB):
        y = jnp.dot(w_ref[...], x_ref[s], preferred_element_type=jnp.float32)
        y = y + b_ref[...]
        y_ref[s] = y
        sm = jnp.sum(y, axis=1, keepdims=True)
        sq = jnp.sum(y * y, axis=1, keepdims=True)
        upd = upd + jnp.concatenate([sm, sq], axis=1)

    @pl.when(i == 0)
    def _():
        st_ref[...] = jnp.zeros_like(st_ref)

    st_ref[...] += upd


def _phase2_one(y1, sc, sh, pr, dwt, db, gp, at, bmt, nb, uwt, ub, gb):
    y1n = y1 * sc + sh
    x2 = jnp.concatenate([y1n, pr], axis=1)
    x2t = x2.T
    lowp = jnp.dot(x2t, dwt, preferred_element_type=jnp.float32)
    lowp = lowp + db
    low = 0.5 * lowp * (1.0 + jax.lax.erf(lowp * 0.7071067811865476))
    res = jnp.dot(low, gp, preferred_element_type=jnp.float32)
    xmt = 0.8 * x2t + 0.2 * res

    rn = jnp.sum(xmt * xmt, axis=1, keepdims=True)
    xnt = xmt / jnp.maximum(jnp.sqrt(rn), 1e-12)
    xsqc = jnp.sum(xnt * xnt, axis=1, keepdims=True)
    xn = xnt.T
    xsqr = jnp.sum(xn * xn, axis=0, keepdims=True)
    gram = jnp.dot(xnt, xn, preferred_element_type=jnp.float32)
    dist = xsqc - 2.0 * gram + xsqr

    u = jnp.dot(xmt, at, preferred_element_type=jnp.float32)
    u = u + nb
    v = jnp.dot(xmt, bmt, preferred_element_type=jnp.float32)

    cif = jax.lax.broadcasted_iota(jnp.int32, (_N, _N), 1).astype(jnp.float32)
    mask = jnp.zeros((_N, _N), jnp.float32)
    its = []
    d = dist
    for _ in range(_K):
        mnv = jnp.min(d, axis=1, keepdims=True)
        it = jnp.min(jnp.where(d == mnv, cif, jnp.inf), axis=1,
                     keepdims=True)
        its.append(it)
        ohf = (cif == it).astype(jnp.float32)
        mask = mask + ohf
        d = jnp.where(ohf != 0.0, jnp.inf, d)

    base = gb * _N
    cols = [it.astype(jnp.int32) + base for it in its]
    selfc = jax.lax.broadcasted_iota(jnp.int32, (_N, 1), 0) + base
    gidx = jnp.concatenate(cols + [selfc] * (16 - _K), axis=1)

    s_g = jnp.dot(mask, v, preferred_element_type=jnp.float32)
    q_g = jnp.dot(mask, v * v, preferred_element_type=jnp.float32)
    lr_s = jnp.dot(mask, low, preferred_element_type=jnp.float32)

    e1 = jnp.sum(_K * u + s_g, axis=0, keepdims=True)
    e2 = jnp.sum(_K * u * u + 2.0 * u * s_g + q_g, axis=0, keepdims=True)
    upd = jnp.concatenate([e1, e2], axis=0)

    ep = jnp.dot(lr_s * (1.0 / _K), uwt,
                 preferred_element_type=jnp.float32) + ub
    return u, v, gidx, upd, ep


def _phase2_body(y1_ref, st_ref, g1_ref, be1_ref, pr_ref, dwt_ref, db_ref,
                 gp_ref, at_ref, bmt_ref, nb_ref, uwt_ref, ub_ref,
                 u_ref, v_ref, gi_ref, acc_ref, ep_ref):
    i = pl.program_id(0)
    n1 = float(_B * _NPIX)
    mu = st_ref[:, 0:1] / n1
    var = st_ref[:, 1:2] / n1 - mu * mu
    sc = g1_ref[...] / jnp.sqrt(var + _EPS)
    sh = be1_ref[...] - mu * sc
    acc = jnp.zeros((2, _C2), jnp.float32)
    for s in range(_NB):
        u, v, gidx, upd, ep = _phase2_one(
            y1_ref[s], sc, sh, pr_ref[...], dwt_ref[...], db_ref[...],
            gp_ref[...], at_ref[...], bmt_ref[...], nb_ref[...],
            uwt_ref[...], ub_ref[...], i * _NB + s)
        u_ref[s] = u
        v_ref[s] = jnp.concatenate(
            [v, jnp.zeros((_N, _C2P - _C2), jnp.float32)], axis=1)
        gi_ref[s] = gidx
        ep_ref[s] = ep
        acc = acc + upd

    @pl.when(i == 0)
    def _():
        acc_ref[...] = jnp.zeros_like(acc_ref)

    acc_ref[...] += acc


def _sc_body(vf, gi, mx, idxv, vb, smx, sv):
    wid = lax.axis_index("c") * 16 + lax.axis_index("s")
    pltpu.sync_copy(gi.at[wid], idxv)

    def body(blk, carry):
        for j in range(_BLK):
            g = blk * _BLK + j
            pltpu.async_copy(vf.at[idxv.at[g]], vb.at[j], sv).wait()
        for j in range(_BLK):
            for c in range(_C2 // 16):
                sl = pl.ds(c * 16, 16)
                m = vb[j, 0, sl]
                for r in range(1, _K):
                    m = jnp.maximum(m, vb[j, r, sl])
                smx[j, sl] = m
        pltpu.sync_copy(smx, mx.at[wid, blk])
        return carry

    lax.fori_loop(0, _NBLK, body, 0)


def _sc_max_gather(vflat, gidx3):
    f32 = jnp.float32
    kern = functools.partial(
        pl.kernel,
        mesh=plsc.VectorSubcoreMesh(core_axis_name="c", subcore_axis_name="s"),
        out_type=jax.ShapeDtypeStruct((_NW, _NBLK, _BLK, _C2P), f32),
        scratch_types=[
            pltpu.VMEM((_RPW, 16), jnp.int32),
            pltpu.VMEM((_BLK, 16, _C2P), f32),
            pltpu.VMEM((_BLK, _C2P), f32),
            pltpu.SemaphoreType.DMA,
        ],
    )(_sc_body)
    return kern(vflat, gidx3)


def _phase3_body(u_ref, mx_ref, acc_ref, g2_ref, be2_ref, w2t_ref,
                 b2_ref, y3_ref, st3_ref):
    i = pl.program_id(0)
    ne = float(_B * _N * _K)
    mu = acc_ref[0:1, :] / ne
    var = acc_ref[1:2, :] / ne - mu * mu
    sc = g2_ref[...] / jnp.sqrt(var + _EPS)
    sh = be2_ref[...] - mu * sc
    upd = jnp.zeros((2, _C), jnp.float32)
    for s in range(_NB):
        z = u_ref[s] + mx_ref[s][:, :_C2]
        g = jnp.maximum(z * sc + sh, 0.0)
        y3 = jnp.dot(g, w2t_ref[...], preferred_element_type=jnp.float32)
        y3 = y3 + b2_ref[...]
        y3_ref[s] = y3
        sm = jnp.sum(y3, axis=0, keepdims=True)
        sq = jnp.sum(y3 * y3, axis=0, keepdims=True)
        upd = upd + jnp.concatenate([sm, sq], axis=0)

    @pl.when(i == 0)
    def _():
        st3_ref[...] = jnp.zeros_like(st3_ref)

    st3_ref[...] += upd


def _phase4_body(y3_ref, ep_ref, st3_ref, g3_ref, be3_ref, x_ref, o_ref):
    n3 = float(_B * _N)
    mu = st3_ref[0:1, :] / n3
    var = st3_ref[1:2, :] / n3 - mu * mu
    sc = g3_ref[...] / jnp.sqrt(var + _EPS)
    sh = be3_ref[...] - mu * sc
    for s in range(_NB):
        o = 0.8 * (y3_ref[s] * sc + sh) + 0.2 * ep_ref[s]
        oc = o[:_NPIX, :]
        o_ref[s] = oc.T + x_ref[s]


def kernel(x, fc1_W, fc1_b, fc1_g, fc1_be, nn_W, nn_b, nn_g, nn_be,
           fc2_W, fc2_b, fc2_g, fc2_be, node_prompts, graph_prompt,
           down_W, down_b, up_W, up_b):
    f32 = jnp.float32
    xf = x.reshape(_B, _C, _NPIX)
    at = (nn_W[:, :_C] - nn_W[:, _C:]).T
    bmt = nn_W[:, _C:].T
    dwt = down_W.T
    w2t = fc2_W.T
    uwt = up_W.T

    y1, st1 = pl.pallas_call(
        _phase1_body,
        grid=(_G,),
        in_specs=[
            pl.BlockSpec((_NB, _C, _NPIX), lambda b: (b, 0, 0)),
            pl.BlockSpec((_C, _C), lambda b: (0, 0)),
            pl.BlockSpec((_C, 1), lambda b: (0, 0)),
        ],
        out_specs=[
            pl.BlockSpec((_NB, _C, _NPIX), lambda b: (b, 0, 0)),
            pl.BlockSpec((_C, 2), lambda b: (0, 0)),
        ],
        out_shape=[
            jax.ShapeDtypeStruct((_B, _C, _NPIX), f32),
            jax.ShapeDtypeStruct((_C, 2), f32),
        ],
    )(xf, fc1_W, fc1_b.reshape(_C, 1))

    u, v, gidx, acc_e, ep = pl.pallas_call(
        _phase2_body,
        grid=(_G,),
        in_specs=[
            pl.BlockSpec((_NB, _C, _NPIX), lambda b: (b, 0, 0)),
            pl.BlockSpec((_C, 2), lambda b: (0, 0)),
            pl.BlockSpec((_C, 1), lambda b: (0, 0)),
            pl.BlockSpec((_C, 1), lambda b: (0, 0)),
            pl.BlockSpec((_C, _P), lambda b: (0, 0)),
            pl.BlockSpec((_C, _R), lambda b: (0, 0)),
            pl.BlockSpec((1, _R), lambda b: (0, 0)),
            pl.BlockSpec((_R, _C), lambda b: (0, 0)),
            pl.BlockSpec((_C, _C2), lambda b: (0, 0)),
            pl.BlockSpec((_C, _C2), lambda b: (0, 0)),
            pl.BlockSpec((1, _C2), lambda b: (0, 0)),
            pl.BlockSpec((_R, _C), lambda b: (0, 0)),
            pl.BlockSpec((1, _C), lambda b: (0, 0)),
        ],
        out_specs=[
            pl.BlockSpec((_NB, _N, _C2), lambda b: (b, 0, 0)),
            pl.BlockSpec((_NB, _N, _C2P), lambda b: (b, 0, 0)),
            pl.BlockSpec((_NB, _N, 16), lambda b: (b, 0, 0)),
            pl.BlockSpec((2, _C2), lambda b: (0, 0)),
            pl.BlockSpec((_NB, _N, _C), lambda b: (b, 0, 0)),
        ],
        out_shape=[
            jax.ShapeDtypeStruct((_B, _N, _C2), f32),
            jax.ShapeDtypeStruct((_B, _N, _C2P), f32),
            jax.ShapeDtypeStruct((_B, _N, 16), jnp.int32),
            jax.ShapeDtypeStruct((2, _C2), f32),
            jax.ShapeDtypeStruct((_B, _N, _C), f32),
        ],
    )(y1, st1, fc1_g.reshape(_C, 1), fc1_be.reshape(_C, 1), node_prompts,
      dwt, down_b.reshape(1, _R), graph_prompt, at, bmt,
      nn_b.reshape(1, _C2), uwt, up_b.reshape(1, _C))

    mx = _sc_max_gather(v.reshape(_ROWS, _C2P), gidx.reshape(_NW, _RPW, 16))
    mx = mx.reshape(_B, _N, _C2P)

    y3, st3 = pl.pallas_call(
        _phase3_body,
        grid=(_G,),
        in_specs=[
            pl.BlockSpec((_NB, _N, _C2), lambda b: (b, 0, 0)),
            pl.BlockSpec((_NB, _N, _C2P), lambda b: (b, 0, 0)),
            pl.BlockSpec((2, _C2), lambda b: (0, 0)),
            pl.BlockSpec((1, _C2), lambda b: (0, 0)),
            pl.BlockSpec((1, _C2), lambda b: (0, 0)),
            pl.BlockSpec((_C2, _C), lambda b: (0, 0)),
            pl.BlockSpec((1, _C), lambda b: (0, 0)),
        ],
        out_specs=[
            pl.BlockSpec((_NB, _N, _C), lambda b: (b, 0, 0)),
            pl.BlockSpec((2, _C), lambda b: (0, 0)),
        ],
        out_shape=[
            jax.ShapeDtypeStruct((_B, _N, _C), f32),
            jax.ShapeDtypeStruct((2, _C), f32),
        ],
    )(u, mx, acc_e, nn_g.reshape(1, _C2), nn_be.reshape(1, _C2),
      w2t, fc2_b.reshape(1, _C))

    out = pl.pallas_call(
        _phase4_body,
        grid=(_G,),
        in_specs=[
            pl.BlockSpec((_NB, _N, _C), lambda b: (b, 0, 0)),
            pl.BlockSpec((_NB, _N, _C), lambda b: (b, 0, 0)),
            pl.BlockSpec((2, _C), lambda b: (0, 0)),
            pl.BlockSpec((1, _C), lambda b: (0, 0)),
            pl.BlockSpec((1, _C), lambda b: (0, 0)),
            pl.BlockSpec((_NB, _C, _NPIX), lambda b: (b, 0, 0)),
        ],
        out_specs=pl.BlockSpec((_NB, _C, _NPIX), lambda b: (b, 0, 0)),
        out_shape=jax.ShapeDtypeStruct((_B, _C, _NPIX), f32),
    )(y3, ep, st3, fc2_g.reshape(1, _C), fc2_be.reshape(1, _C), xf)

    return out.reshape(_B, _C, _H, _W)


# speedup vs baseline: 1.9445x; 1.9445x over previous
import functools

import jax
import jax.numpy as jnp
from jax import lax
from jax.experimental import pallas as pl
from jax.experimental.pallas import tpu as pltpu
from jax.experimental.pallas import tpu_sc as plsc

_B, _C, _H, _W = 64, 96, 14, 14
_P = 14
_NPIX = _H * _W
_N = _NPIX + _P
_K = 9
_R = 32
_C2 = 2 * _C
_EPS = 1e-5
_NB = 8
_G = _B // _NB

_ROWS = _B * _N
_NW = 32
_RPW = _ROWS // _NW
_BLK = 6
_NBLK = _RPW // _BLK
_C2P = 256


def _phase1_body(x_ref, w_ref, b_ref, y_ref, st_ref):
    i = pl.program_id(0)
    upd = jnp.zeros((_C, 2), jnp.float32)
    for s in range(_NB):
        y = jnp.dot(w_ref[...], x_ref[s], preferred_element_type=jnp.float32)
        y = y + b_ref[...]
        y_ref[s] = y
        sm = jnp.sum(y, axis=1, keepdims=True)
        sq = jnp.sum(y * y, axis=1, keepdims=True)
        upd = upd + jnp.concatenate([sm, sq], axis=1)

    @pl.when(i == 0)
    def _():
        st_ref[...] = jnp.zeros_like(st_ref)

    st_ref[...] += upd


def _phase2_one(y1, sc, sh, pr, dwt, db, gp, at, bmt, nb, uwt, ub, gb):
    y1n = y1 * sc + sh
    x2 = jnp.concatenate([y1n, pr], axis=1)
    x2t = x2.T
    lowp = jnp.dot(x2t, dwt, preferred_element_type=jnp.float32)
    lowp = lowp + db
    low = 0.5 * lowp * (1.0 + jax.lax.erf(lowp * 0.7071067811865476))
    res = jnp.dot(low, gp, preferred_element_type=jnp.float32)
    xmt = 0.8 * x2t + 0.2 * res

    rn = jnp.sum(xmt * xmt, axis=1, keepdims=True)
    xnt = xmt / jnp.maximum(jnp.sqrt(rn), 1e-12)
    xsqc = jnp.sum(xnt * xnt, axis=1, keepdims=True)
    xn = xnt.T
    xsqr = jnp.sum(xn * xn, axis=0, keepdims=True)
    gram = jnp.dot(xnt, xn, preferred_element_type=jnp.float32)
    dist = xsqc - 2.0 * gram + xsqr

    u = jnp.dot(xmt, at, preferred_element_type=jnp.float32)
    u = u + nb
    v = jnp.dot(xmt, bmt, preferred_element_type=jnp.float32)

    cif = jax.lax.broadcasted_iota(jnp.int32, (_N, _N), 1).astype(jnp.float32)
    mask = jnp.zeros((_N, _N), jnp.float32)
    its = []
    d = dist
    for _ in range(_K):
        mnv = jnp.min(d, axis=1, keepdims=True)
        it = jnp.min(jnp.where(d == mnv, cif, jnp.inf), axis=1,
                     keepdims=True)
        its.append(it)
        ohf = (cif == it).astype(jnp.float32)
        mask = mask + ohf
        d = jnp.where(ohf != 0.0, jnp.inf, d)

    base = gb * _N
    cols = [it.astype(jnp.int32) + base for it in its]
    selfc = jax.lax.broadcasted_iota(jnp.int32, (_N, 1), 0) + base
    gidx = jnp.concatenate(cols + [selfc] * (16 - _K), axis=1)

    s_g = jnp.dot(mask, v, preferred_element_type=jnp.float32)
    q_g = jnp.dot(mask, v * v, preferred_element_type=jnp.float32)
    lr_s = jnp.dot(mask, low, preferred_element_type=jnp.float32)

    e1 = jnp.sum(_K * u + s_g, axis=0, keepdims=True)
    e2 = jnp.sum(_K * u * u + 2.0 * u * s_g + q_g, axis=0, keepdims=True)
    upd = jnp.concatenate([e1, e2], axis=0)

    ep = jnp.dot(lr_s * (1.0 / _K), uwt,
                 preferred_element_type=jnp.float32) + ub
    return u, v, gidx, upd, ep


def _phase2_body(y1_ref, st_ref, g1_ref, be1_ref, pr_ref, dwt_ref, db_ref,
                 gp_ref, at_ref, bmt_ref, nb_ref, uwt_ref, ub_ref,
                 u_ref, v_ref, gi_ref, acc_ref, ep_ref):
    i = pl.program_id(0)
    n1 = float(_B * _NPIX)
    mu = st_ref[:, 0:1] / n1
    var = st_ref[:, 1:2] / n1 - mu * mu
    sc = g1_ref[...] / jnp.sqrt(var + _EPS)
    sh = be1_ref[...] - mu * sc
    acc = jnp.zeros((2, _C2), jnp.float32)
    for s in range(_NB):
        u, v, gidx, upd, ep = _phase2_one(
            y1_ref[s], sc, sh, pr_ref[...], dwt_ref[...], db_ref[...],
            gp_ref[...], at_ref[...], bmt_ref[...], nb_ref[...],
            uwt_ref[...], ub_ref[...], i * _NB + s)
        u_ref[s] = u
        v_ref[s] = jnp.concatenate(
            [v, jnp.zeros((_N, _C2P - _C2), jnp.float32)], axis=1)
        gi_ref[s] = gidx
        ep_ref[s] = ep
        acc = acc + upd

    @pl.when(i == 0)
    def _():
        acc_ref[...] = jnp.zeros_like(acc_ref)

    acc_ref[...] += acc


def _sc_body(vf, gi, mx, idxv, vb0, vb1, smx, sv0, sv1):
    wid = lax.axis_index("c") * 16 + lax.axis_index("s")
    pltpu.sync_copy(gi.at[wid], idxv)

    def fire(blk, vb, sv):
        for j in range(_BLK):
            g = blk * _BLK + j
            pltpu.async_copy(vf.at[idxv.at[g]], vb.at[j], sv)

    def drain(blk, vb, sv):
        for j in range(_BLK):
            g = blk * _BLK + j
            pltpu.make_async_copy(vf.at[idxv.at[g]], vb.at[j], sv).wait()

    def compute(blk, vb):
        for j in range(_BLK):
            for c in range(_C2 // 16):
                sl = pl.ds(c * 16, 16)
                m = vb[j, 0, sl]
                for r in range(1, _K):
                    m = jnp.maximum(m, vb[j, r, sl])
                smx[j, sl] = m
        pltpu.sync_copy(smx, mx.at[wid, blk])

    fire(0, vb0, sv0)

    def body(pair, carry):
        blk0 = 2 * pair
        blk1 = blk0 + 1
        fire(blk1, vb1, sv1)
        drain(blk0, vb0, sv0)
        compute(blk0, vb0)

        @pl.when(blk0 + 2 < _NBLK)
        def _():
            fire(blk0 + 2, vb0, sv0)

        drain(blk1, vb1, sv1)
        compute(blk1, vb1)
        return carry

    lax.fori_loop(0, _NBLK // 2, body, 0)


def _sc_max_gather(vflat, gidx3):
    f32 = jnp.float32
    kern = functools.partial(
        pl.kernel,
        mesh=plsc.VectorSubcoreMesh(core_axis_name="c", subcore_axis_name="s"),
        out_type=jax.ShapeDtypeStruct((_NW, _NBLK, _BLK, _C2P), f32),
        scratch_types=[
            pltpu.VMEM((_RPW, 16), jnp.int32),
            pltpu.VMEM((_BLK, 16, _C2P), f32),
            pltpu.VMEM((_BLK, 16, _C2P), f32),
            pltpu.VMEM((_BLK, _C2P), f32),
            pltpu.SemaphoreType.DMA,
            pltpu.SemaphoreType.DMA,
        ],
    )(_sc_body)
    return kern(vflat, gidx3)


def _phase3_body(u_ref, mx_ref, acc_ref, g2_ref, be2_ref, w2t_ref,
                 b2_ref, y3_ref, st3_ref):
    i = pl.program_id(0)
    ne = float(_B * _N * _K)
    mu = acc_ref[0:1, :] / ne
    var = acc_ref[1:2, :] / ne - mu * mu
    sc = g2_ref[...] / jnp.sqrt(var + _EPS)
    sh = be2_ref[...] - mu * sc
    upd = jnp.zeros((2, _C), jnp.float32)
    for s in range(_NB):
        z = u_ref[s] + mx_ref[s][:, :_C2]
        g = jnp.maximum(z * sc + sh, 0.0)
        y3 = jnp.dot(g, w2t_ref[...], preferred_element_type=jnp.float32)
        y3 = y3 + b2_ref[...]
        y3_ref[s] = y3
        sm = jnp.sum(y3, axis=0, keepdims=True)
        sq = jnp.sum(y3 * y3, axis=0, keepdims=True)
        upd = upd + jnp.concatenate([sm, sq], axis=0)

    @pl.when(i == 0)
    def _():
        st3_ref[...] = jnp.zeros_like(st3_ref)

    st3_ref[...] += upd


def _phase4_body(y3_ref, ep_ref, st3_ref, g3_ref, be3_ref, x_ref, o_ref):
    n3 = float(_B * _N)
    mu = st3_ref[0:1, :] / n3
    var = st3_ref[1:2, :] / n3 - mu * mu
    sc = g3_ref[...] / jnp.sqrt(var + _EPS)
    sh = be3_ref[...] - mu * sc
    for s in range(_NB):
        o = 0.8 * (y3_ref[s] * sc + sh) + 0.2 * ep_ref[s]
        oc = o[:_NPIX, :]
        o_ref[s] = oc.T + x_ref[s]


def kernel(x, fc1_W, fc1_b, fc1_g, fc1_be, nn_W, nn_b, nn_g, nn_be,
           fc2_W, fc2_b, fc2_g, fc2_be, node_prompts, graph_prompt,
           down_W, down_b, up_W, up_b):
    f32 = jnp.float32
    xf = x.reshape(_B, _C, _NPIX)
    at = (nn_W[:, :_C] - nn_W[:, _C:]).T
    bmt = nn_W[:, _C:].T
    dwt = down_W.T
    w2t = fc2_W.T
    uwt = up_W.T

    y1, st1 = pl.pallas_call(
        _phase1_body,
        grid=(_G,),
        in_specs=[
            pl.BlockSpec((_NB, _C, _NPIX), lambda b: (b, 0, 0)),
            pl.BlockSpec((_C, _C), lambda b: (0, 0)),
            pl.BlockSpec((_C, 1), lambda b: (0, 0)),
        ],
        out_specs=[
            pl.BlockSpec((_NB, _C, _NPIX), lambda b: (b, 0, 0)),
            pl.BlockSpec((_C, 2), lambda b: (0, 0)),
        ],
        out_shape=[
            jax.ShapeDtypeStruct((_B, _C, _NPIX), f32),
            jax.ShapeDtypeStruct((_C, 2), f32),
        ],
    )(xf, fc1_W, fc1_b.reshape(_C, 1))

    u, v, gidx, acc_e, ep = pl.pallas_call(
        _phase2_body,
        grid=(_G,),
        in_specs=[
            pl.BlockSpec((_NB, _C, _NPIX), lambda b: (b, 0, 0)),
            pl.BlockSpec((_C, 2), lambda b: (0, 0)),
            pl.BlockSpec((_C, 1), lambda b: (0, 0)),
            pl.BlockSpec((_C, 1), lambda b: (0, 0)),
            pl.BlockSpec((_C, _P), lambda b: (0, 0)),
            pl.BlockSpec((_C, _R), lambda b: (0, 0)),
            pl.BlockSpec((1, _R), lambda b: (0, 0)),
            pl.BlockSpec((_R, _C), lambda b: (0, 0)),
            pl.BlockSpec((_C, _C2), lambda b: (0, 0)),
            pl.BlockSpec((_C, _C2), lambda b: (0, 0)),
            pl.BlockSpec((1, _C2), lambda b: (0, 0)),
            pl.BlockSpec((_R, _C), lambda b: (0, 0)),
            pl.BlockSpec((1, _C), lambda b: (0, 0)),
        ],
        out_specs=[
            pl.BlockSpec((_NB, _N, _C2), lambda b: (b, 0, 0)),
            pl.BlockSpec((_NB, _N, _C2P), lambda b: (b, 0, 0)),
            pl.BlockSpec((_NB, _N, 16), lambda b: (b, 0, 0)),
            pl.BlockSpec((2, _C2), lambda b: (0, 0)),
            pl.BlockSpec((_NB, _N, _C), lambda b: (b, 0, 0)),
        ],
        out_shape=[
            jax.ShapeDtypeStruct((_B, _N, _C2), f32),
            jax.ShapeDtypeStruct((_B, _N, _C2P), f32),
            jax.ShapeDtypeStruct((_B, _N, 16), jnp.int32),
            jax.ShapeDtypeStruct((2, _C2), f32),
            jax.ShapeDtypeStruct((_B, _N, _C), f32),
        ],
    )(y1, st1, fc1_g.reshape(_C, 1), fc1_be.reshape(_C, 1), node_prompts,
      dwt, down_b.reshape(1, _R), graph_prompt, at, bmt,
      nn_b.reshape(1, _C2), uwt, up_b.reshape(1, _C))

    mx = _sc_max_gather(v.reshape(_ROWS, _C2P), gidx.reshape(_NW, _RPW, 16))
    mx = mx.reshape(_B, _N, _C2P)

    y3, st3 = pl.pallas_call(
        _phase3_body,
        grid=(_G,),
        in_specs=[
            pl.BlockSpec((_NB, _N, _C2), lambda b: (b, 0, 0)),
            pl.BlockSpec((_NB, _N, _C2P), lambda b: (b, 0, 0)),
            pl.BlockSpec((2, _C2), lambda b: (0, 0)),
            pl.BlockSpec((1, _C2), lambda b: (0, 0)),
            pl.BlockSpec((1, _C2), lambda b: (0, 0)),
            pl.BlockSpec((_C2, _C), lambda b: (0, 0)),
            pl.BlockSpec((1, _C), lambda b: (0, 0)),
        ],
        out_specs=[
            pl.BlockSpec((_NB, _N, _C), lambda b: (b, 0, 0)),
            pl.BlockSpec((2, _C), lambda b: (0, 0)),
        ],
        out_shape=[
            jax.ShapeDtypeStruct((_B, _N, _C), f32),
            jax.ShapeDtypeStruct((2, _C), f32),
        ],
    )(u, mx, acc_e, nn_g.reshape(1, _C2), nn_be.reshape(1, _C2),
      w2t, fc2_b.reshape(1, _C))

    out = pl.pallas_call(
        _phase4_body,
        grid=(_G,),
        in_specs=[
            pl.BlockSpec((_NB, _N, _C), lambda b: (b, 0, 0)),
            pl.BlockSpec((_NB, _N, _C), lambda b: (b, 0, 0)),
            pl.BlockSpec((2, _C), lambda b: (0, 0)),
            pl.BlockSpec((1, _C), lambda b: (0, 0)),
            pl.BlockSpec((1, _C), lambda b: (0, 0)),
            pl.BlockSpec((_NB, _C, _NPIX), lambda b: (b, 0, 0)),
        ],
        out_specs=pl.BlockSpec((_NB, _C, _NPIX), lambda b: (b, 0, 0)),
        out_shape=jax.ShapeDtypeStruct((_B, _C, _NPIX), f32),
    )(y3, ep, st3, fc2_g.reshape(1, _C), fc2_be.reshape(1, _C), xf)

    return out.reshape(_B, _C, _H, _W)


# speedup vs baseline: 1.9581x; 1.0070x over previous
import functools

import jax
import jax.numpy as jnp
from jax import lax
from jax.experimental import pallas as pl
from jax.experimental.pallas import tpu as pltpu
from jax.experimental.pallas import tpu_sc as plsc

_B, _C, _H, _W = 64, 96, 14, 14
_P = 14
_NPIX = _H * _W
_N = _NPIX + _P
_K = 9
_R = 32
_C2 = 2 * _C
_EPS = 1e-5
_NB = 8
_G = _B // _NB

_ROWS = _B * _N
_NW = 32
_RPW = _ROWS // _NW
_BLK = 6
_NBLK = _RPW // _BLK
_C2P = 256


def _phase1_body(x_ref, w_ref, b_ref, y_ref, st_ref):
    i = pl.program_id(0)
    upd = jnp.zeros((_C, 2), jnp.float32)
    for s in range(_NB):
        y = jnp.dot(w_ref[...], x_ref[s], preferred_element_type=jnp.float32)
        y = y + b_ref[...]
        y_ref[s] = y
        sm = jnp.sum(y, axis=1, keepdims=True)
        sq = jnp.sum(y * y, axis=1, keepdims=True)
        upd = upd + jnp.concatenate([sm, sq], axis=1)

    @pl.when(i == 0)
    def _():
        st_ref[...] = jnp.zeros_like(st_ref)

    st_ref[...] += upd


def _phase2_one(y1, sc, sh, pr, dwt, db, gp, at, bmt, nb, uwt, ub, gb):
    y1n = y1 * sc + sh
    x2 = jnp.concatenate([y1n, pr], axis=1)
    x2t = x2.T
    lowp = jnp.dot(x2t, dwt, preferred_element_type=jnp.float32)
    lowp = lowp + db
    low = 0.5 * lowp * (1.0 + jax.lax.erf(lowp * 0.7071067811865476))
    res = jnp.dot(low, gp, preferred_element_type=jnp.float32)
    xmt = 0.8 * x2t + 0.2 * res

    rn = jnp.sum(xmt * xmt, axis=1, keepdims=True)
    xnt = xmt / jnp.maximum(jnp.sqrt(rn), 1e-12)
    xsqc = jnp.sum(xnt * xnt, axis=1, keepdims=True)
    xn = xnt.T
    xsqr = jnp.sum(xn * xn, axis=0, keepdims=True)
    gram = jnp.dot(xnt, xn, preferred_element_type=jnp.float32)
    dist = xsqc - 2.0 * gram + xsqr

    u = jnp.dot(xmt, at, preferred_element_type=jnp.float32)
    u = u + nb
    v = jnp.dot(xmt, bmt, preferred_element_type=jnp.float32)

    cif = jax.lax.broadcasted_iota(jnp.int32, (_N, _N), 1).astype(jnp.float32)
    mask = jnp.zeros((_N, _N), jnp.float32)
    its = []
    d = dist
    for _ in range(_K):
        mnv = jnp.min(d, axis=1, keepdims=True)
        it = jnp.min(jnp.where(d == mnv, cif, jnp.inf), axis=1,
                     keepdims=True)
        its.append(it)
        ohf = (cif == it).astype(jnp.float32)
        mask = mask + ohf
        d = jnp.where(ohf != 0.0, jnp.inf, d)

    base = gb * _N
    cols = [it.astype(jnp.int32) + base for it in its]
    selfc = jax.lax.broadcasted_iota(jnp.int32, (_N, 1), 0) + base
    gidx = jnp.concatenate(cols + [selfc] * (16 - _K), axis=1)

    s_g = jnp.dot(mask, v, preferred_element_type=jnp.float32)
    q_g = jnp.dot(mask, v * v, preferred_element_type=jnp.float32)
    lr_s = jnp.dot(mask, low, preferred_element_type=jnp.float32)

    e1 = jnp.sum(_K * u + s_g, axis=0, keepdims=True)
    e2 = jnp.sum(_K * u * u + 2.0 * u * s_g + q_g, axis=0, keepdims=True)
    upd = jnp.concatenate([e1, e2], axis=0)

    ep = jnp.dot(lr_s * (1.0 / _K), uwt,
                 preferred_element_type=jnp.float32) + ub
    return u, v, gidx, upd, ep


def _phase2_body(y1_ref, st_ref, g1_ref, be1_ref, pr_ref, dwt_ref, db_ref,
                 gp_ref, at_ref, bmt_ref, nb_ref, uwt_ref, ub_ref,
                 u_ref, v_ref, gi_ref, acc_ref, ep_ref):
    i = pl.program_id(0)
    n1 = float(_B * _NPIX)
    mu = st_ref[:, 0:1] / n1
    var = st_ref[:, 1:2] / n1 - mu * mu
    sc = g1_ref[...] / jnp.sqrt(var + _EPS)
    sh = be1_ref[...] - mu * sc
    acc = jnp.zeros((2, _C2), jnp.float32)
    for s in range(_NB):
        u, v, gidx, upd, ep = _phase2_one(
            y1_ref[s], sc, sh, pr_ref[...], dwt_ref[...], db_ref[...],
            gp_ref[...], at_ref[...], bmt_ref[...], nb_ref[...],
            uwt_ref[...], ub_ref[...], i * _NB + s)
        u_ref[s] = u
        v_ref[s] = jnp.concatenate(
            [v, jnp.zeros((_N, _C2P - _C2), jnp.float32)], axis=1)
        gi_ref[s] = gidx
        ep_ref[s] = ep
        acc = acc + upd

    @pl.when(i == 0)
    def _():
        acc_ref[...] = jnp.zeros_like(acc_ref)

    acc_ref[...] += acc


def _sc_body(vf, gi, mx, idxv, vb0, vb1, smx, sv0, sv1):
    wid = lax.axis_index("c") * 16 + lax.axis_index("s")
    pltpu.sync_copy(gi.at[wid], idxv)

    def fire(blk, vb, sv):
        pltpu.async_copy(vf.at[idxv.at[pl.ds(blk * _BLK * 16, _BLK * 16)]],
                         vb, sv)

    def drain(blk, vb, sv):
        pltpu.make_async_copy(
            vf.at[idxv.at[pl.ds(blk * _BLK * 16, _BLK * 16)]], vb, sv).wait()

    def compute(blk, vb):
        for j in range(_BLK):
            for c in range(_C2 // 16):
                sl = pl.ds(c * 16, 16)
                m = vb[j * 16, sl]
                for r in range(1, _K):
                    m = jnp.maximum(m, vb[j * 16 + r, sl])
                smx[j, sl] = m
        pltpu.sync_copy(smx, mx.at[wid, blk])

    fire(0, vb0, sv0)

    def body(pair, carry):
        blk0 = 2 * pair
        blk1 = blk0 + 1
        fire(blk1, vb1, sv1)
        drain(blk0, vb0, sv0)
        compute(blk0, vb0)

        @pl.when(blk0 + 2 < _NBLK)
        def _():
            fire(blk0 + 2, vb0, sv0)

        drain(blk1, vb1, sv1)
        compute(blk1, vb1)
        return carry

    lax.fori_loop(0, _NBLK // 2, body, 0)


def _sc_max_gather(vflat, gidx2):
    f32 = jnp.float32
    kern = functools.partial(
        pl.kernel,
        mesh=plsc.VectorSubcoreMesh(core_axis_name="c", subcore_axis_name="s"),
        out_type=jax.ShapeDtypeStruct((_NW, _NBLK, _BLK, _C2P), f32),
        scratch_types=[
            pltpu.VMEM((_RPW * 16,), jnp.int32),
            pltpu.VMEM((_BLK * 16, _C2P), f32),
            pltpu.VMEM((_BLK * 16, _C2P), f32),
            pltpu.VMEM((_BLK, _C2P), f32),
            pltpu.SemaphoreType.DMA,
            pltpu.SemaphoreType.DMA,
        ],
    )(_sc_body)
    return kern(vflat, gidx2)


def _phase3_body(u_ref, mx_ref, acc_ref, g2_ref, be2_ref, w2t_ref,
                 b2_ref, y3_ref, st3_ref):
    i = pl.program_id(0)
    ne = float(_B * _N * _K)
    mu = acc_ref[0:1, :] / ne
    var = acc_ref[1:2, :] / ne - mu * mu
    sc = g2_ref[...] / jnp.sqrt(var + _EPS)
    sh = be2_ref[...] - mu * sc
    upd = jnp.zeros((2, _C), jnp.float32)
    for s in range(_NB):
        z = u_ref[s] + mx_ref[s][:, :_C2]
        g = jnp.maximum(z * sc + sh, 0.0)
        y3 = jnp.dot(g, w2t_ref[...], preferred_element_type=jnp.float32)
        y3 = y3 + b2_ref[...]
        y3_ref[s] = y3
        sm = jnp.sum(y3, axis=0, keepdims=True)
        sq = jnp.sum(y3 * y3, axis=0, keepdims=True)
        upd = upd + jnp.concatenate([sm, sq], axis=0)

    @pl.when(i == 0)
    def _():
        st3_ref[...] = jnp.zeros_like(st3_ref)

    st3_ref[...] += upd


def _phase4_body(y3_ref, ep_ref, st3_ref, g3_ref, be3_ref, x_ref, o_ref):
    n3 = float(_B * _N)
    mu = st3_ref[0:1, :] / n3
    var = st3_ref[1:2, :] / n3 - mu * mu
    sc = g3_ref[...] / jnp.sqrt(var + _EPS)
    sh = be3_ref[...] - mu * sc
    for s in range(_NB):
        o = 0.8 * (y3_ref[s] * sc + sh) + 0.2 * ep_ref[s]
        oc = o[:_NPIX, :]
        o_ref[s] = oc.T + x_ref[s]


def kernel(x, fc1_W, fc1_b, fc1_g, fc1_be, nn_W, nn_b, nn_g, nn_be,
           fc2_W, fc2_b, fc2_g, fc2_be, node_prompts, graph_prompt,
           down_W, down_b, up_W, up_b):
    f32 = jnp.float32
    xf = x.reshape(_B, _C, _NPIX)
    at = (nn_W[:, :_C] - nn_W[:, _C:]).T
    bmt = nn_W[:, _C:].T
    dwt = down_W.T
    w2t = fc2_W.T
    uwt = up_W.T

    y1, st1 = pl.pallas_call(
        _phase1_body,
        grid=(_G,),
        in_specs=[
            pl.BlockSpec((_NB, _C, _NPIX), lambda b: (b, 0, 0)),
            pl.BlockSpec((_C, _C), lambda b: (0, 0)),
            pl.BlockSpec((_C, 1), lambda b: (0, 0)),
        ],
        out_specs=[
            pl.BlockSpec((_NB, _C, _NPIX), lambda b: (b, 0, 0)),
            pl.BlockSpec((_C, 2), lambda b: (0, 0)),
        ],
        out_shape=[
            jax.ShapeDtypeStruct((_B, _C, _NPIX), f32),
            jax.ShapeDtypeStruct((_C, 2), f32),
        ],
    )(xf, fc1_W, fc1_b.reshape(_C, 1))

    u, v, gidx, acc_e, ep = pl.pallas_call(
        _phase2_body,
        grid=(_G,),
        in_specs=[
            pl.BlockSpec((_NB, _C, _NPIX), lambda b: (b, 0, 0)),
            pl.BlockSpec((_C, 2), lambda b: (0, 0)),
            pl.BlockSpec((_C, 1), lambda b: (0, 0)),
            pl.BlockSpec((_C, 1), lambda b: (0, 0)),
            pl.BlockSpec((_C, _P), lambda b: (0, 0)),
            pl.BlockSpec((_C, _R), lambda b: (0, 0)),
            pl.BlockSpec((1, _R), lambda b: (0, 0)),
            pl.BlockSpec((_R, _C), lambda b: (0, 0)),
            pl.BlockSpec((_C, _C2), lambda b: (0, 0)),
            pl.BlockSpec((_C, _C2), lambda b: (0, 0)),
            pl.BlockSpec((1, _C2), lambda b: (0, 0)),
            pl.BlockSpec((_R, _C), lambda b: (0, 0)),
            pl.BlockSpec((1, _C), lambda b: (0, 0)),
        ],
        out_specs=[
            pl.BlockSpec((_NB, _N, _C2), lambda b: (b, 0, 0)),
            pl.BlockSpec((_NB, _N, _C2P), lambda b: (b, 0, 0)),
            pl.BlockSpec((_NB, _N, 16), lambda b: (b, 0, 0)),
            pl.BlockSpec((2, _C2), lambda b: (0, 0)),
            pl.BlockSpec((_NB, _N, _C), lambda b: (b, 0, 0)),
        ],
        out_shape=[
            jax.ShapeDtypeStruct((_B, _N, _C2), f32),
            jax.ShapeDtypeStruct((_B, _N, _C2P), f32),
            jax.ShapeDtypeStruct((_B, _N, 16), jnp.int32),
            jax.ShapeDtypeStruct((2, _C2), f32),
            jax.ShapeDtypeStruct((_B, _N, _C), f32),
        ],
    )(y1, st1, fc1_g.reshape(_C, 1), fc1_be.reshape(_C, 1), node_prompts,
      dwt, down_b.reshape(1, _R), graph_prompt, at, bmt,
      nn_b.reshape(1, _C2), uwt, up_b.reshape(1, _C))

    mx = _sc_max_gather(v.reshape(_ROWS, _C2P), gidx.reshape(_NW, _RPW * 16))
    mx = mx.reshape(_B, _N, _C2P)

    y3, st3 = pl.pallas_call(
        _phase3_body,
        grid=(_G,),
        in_specs=[
            pl.BlockSpec((_NB, _N, _C2), lambda b: (b, 0, 0)),
            pl.BlockSpec((_NB, _N, _C2P), lambda b: (b, 0, 0)),
            pl.BlockSpec((2, _C2), lambda b: (0, 0)),
            pl.BlockSpec((1, _C2), lambda b: (0, 0)),
            pl.BlockSpec((1, _C2), lambda b: (0, 0)),
            pl.BlockSpec((_C2, _C), lambda b: (0, 0)),
            pl.BlockSpec((1, _C), lambda b: (0, 0)),
        ],
        out_specs=[
            pl.BlockSpec((_NB, _N, _C), lambda b: (b, 0, 0)),
            pl.BlockSpec((2, _C), lambda b: (0, 0)),
        ],
        out_shape=[
            jax.ShapeDtypeStruct((_B, _N, _C), f32),
            jax.ShapeDtypeStruct((2, _C), f32),
        ],
    )(u, mx, acc_e, nn_g.reshape(1, _C2), nn_be.reshape(1, _C2),
      w2t, fc2_b.reshape(1, _C))

    out = pl.pallas_call(
        _phase4_body,
        grid=(_G,),
        in_specs=[
            pl.BlockSpec((_NB, _N, _C), lambda b: (b, 0, 0)),
            pl.BlockSpec((_NB, _N, _C), lambda b: (b, 0, 0)),
            pl.BlockSpec((2, _C), lambda b: (0, 0)),
            pl.BlockSpec((1, _C), lambda b: (0, 0)),
            pl.BlockSpec((1, _C), lambda b: (0, 0)),
            pl.BlockSpec((_NB, _C, _NPIX), lambda b: (b, 0, 0)),
        ],
        out_specs=pl.BlockSpec((_NB, _C, _NPIX), lambda b: (b, 0, 0)),
        out_shape=jax.ShapeDtypeStruct((_B, _C, _NPIX), f32),
    )(y3, ep, st3, fc2_g.reshape(1, _C), fc2_be.reshape(1, _C), xf)

    return out.reshape(_B, _C, _H, _W)


# speedup vs baseline: 1.9839x; 1.0132x over previous
import functools

import jax
import jax.numpy as jnp
from jax import lax
from jax.experimental import pallas as pl
from jax.experimental.pallas import tpu as pltpu
from jax.experimental.pallas import tpu_sc as plsc

_B, _C, _H, _W = 64, 96, 14, 14
_P = 14
_NPIX = _H * _W
_N = _NPIX + _P
_K = 9
_R = 32
_C2 = 2 * _C
_EPS = 1e-5
_NB = 8
_G = _B // _NB

_ROWS = _B * _N
_NWU = 30
_RPW = _ROWS // _NWU
_BLKN = 8
_NBLK = _RPW // _BLKN
_C2P = 256


def _phase1_body(x_ref, w_ref, b_ref, y_ref, st_ref):
    i = pl.program_id(0)
    upd = jnp.zeros((_C, 2), jnp.float32)
    for s in range(_NB):
        y = jnp.dot(w_ref[...], x_ref[s], preferred_element_type=jnp.float32)
        y = y + b_ref[...]
        y_ref[s] = y
        sm = jnp.sum(y, axis=1, keepdims=True)
        sq = jnp.sum(y * y, axis=1, keepdims=True)
        upd = upd + jnp.concatenate([sm, sq], axis=1)

    @pl.when(i == 0)
    def _():
        st_ref[...] = jnp.zeros_like(st_ref)

    st_ref[...] += upd


def _phase2_one(y1, sc, sh, pr, dwt, db, gp, at, bmt, nb, uwt, ub, gb):
    y1n = y1 * sc + sh
    x2 = jnp.concatenate([y1n, pr], axis=1)
    x2t = x2.T
    lowp = jnp.dot(x2t, dwt, preferred_element_type=jnp.float32)
    lowp = lowp + db
    low = 0.5 * lowp * (1.0 + jax.lax.erf(lowp * 0.7071067811865476))
    res = jnp.dot(low, gp, preferred_element_type=jnp.float32)
    xmt = 0.8 * x2t + 0.2 * res

    rn = jnp.sum(xmt * xmt, axis=1, keepdims=True)
    xnt = xmt / jnp.maximum(jnp.sqrt(rn), 1e-12)
    xsqc = jnp.sum(xnt * xnt, axis=1, keepdims=True)
    xn = xnt.T
    xsqr = jnp.sum(xn * xn, axis=0, keepdims=True)
    gram = jnp.dot(xnt, xn, preferred_element_type=jnp.float32)
    dist = xsqc - 2.0 * gram + xsqr

    u = jnp.dot(xmt, at, preferred_element_type=jnp.float32)
    u = u + nb
    v = jnp.dot(xmt, bmt, preferred_element_type=jnp.float32)

    cif = jax.lax.broadcasted_iota(jnp.int32, (_N, _N), 1).astype(jnp.float32)
    mask = jnp.zeros((_N, _N), jnp.float32)
    its = []
    d = dist
    for _ in range(_K):
        mnv = jnp.min(d, axis=1, keepdims=True)
        it = jnp.min(jnp.where(d == mnv, cif, jnp.inf), axis=1,
                     keepdims=True)
        its.append(it)
        ohf = (cif == it).astype(jnp.float32)
        mask = mask + ohf
        d = jnp.where(ohf != 0.0, jnp.inf, d)

    base = gb * _N
    cols = [it.astype(jnp.int32) + base for it in its]
    gidx = jnp.concatenate(cols, axis=1)

    s_g = jnp.dot(mask, v, preferred_element_type=jnp.float32)
    q_g = jnp.dot(mask, v * v, preferred_element_type=jnp.float32)
    lr_s = jnp.dot(mask, low, preferred_element_type=jnp.float32)

    e1 = jnp.sum(_K * u + s_g, axis=0, keepdims=True)
    e2 = jnp.sum(_K * u * u + 2.0 * u * s_g + q_g, axis=0, keepdims=True)
    upd = jnp.concatenate([e1, e2], axis=0)

    ep = jnp.dot(lr_s * (1.0 / _K), uwt,
                 preferred_element_type=jnp.float32) + ub
    return u, v, gidx, upd, ep


def _phase2_body(y1_ref, st_ref, g1_ref, be1_ref, pr_ref, dwt_ref, db_ref,
                 gp_ref, at_ref, bmt_ref, nb_ref, uwt_ref, ub_ref,
                 u_ref, v_ref, gi_ref, acc_ref, ep_ref):
    i = pl.program_id(0)
    n1 = float(_B * _NPIX)
    mu = st_ref[:, 0:1] / n1
    var = st_ref[:, 1:2] / n1 - mu * mu
    sc = g1_ref[...] / jnp.sqrt(var + _EPS)
    sh = be1_ref[...] - mu * sc
    acc = jnp.zeros((2, _C2), jnp.float32)
    for s in range(_NB):
        u, v, gidx, upd, ep = _phase2_one(
            y1_ref[s], sc, sh, pr_ref[...], dwt_ref[...], db_ref[...],
            gp_ref[...], at_ref[...], bmt_ref[...], nb_ref[...],
            uwt_ref[...], ub_ref[...], i * _NB + s)
        u_ref[s] = u
        v_ref[s] = jnp.concatenate(
            [v, jnp.zeros((_N, _C2P - _C2), jnp.float32)], axis=1)
        gi_ref[s] = gidx
        ep_ref[s] = ep
        acc = acc + upd

    @pl.when(i == 0)
    def _():
        acc_ref[...] = jnp.zeros_like(acc_ref)

    acc_ref[...] += acc


def _sc_body(vf, gi, mx, idxv, vb0, vb1, smx, sv0, sv1):
    wid = lax.axis_index("c") * 16 + lax.axis_index("s")

    @pl.when(wid < _NWU)
    def _():
        nidx = _RPW * _K
        bidx = _BLKN * _K
        pltpu.sync_copy(gi.at[pl.ds(wid * nidx, nidx)], idxv)

        def fire(blk, vb, sv):
            pltpu.async_copy(vf.at[idxv.at[pl.ds(blk * bidx, bidx)]], vb, sv)

        def drain(blk, vb, sv):
            pltpu.make_async_copy(
                vf.at[idxv.at[pl.ds(blk * bidx, bidx)]], vb, sv).wait()

        def compute(blk, vb):
            for j in range(_BLKN):
                for c in range(_C2 // 16):
                    sl = pl.ds(c * 16, 16)
                    m = vb[j * _K, sl]
                    for r in range(1, _K):
                        m = jnp.maximum(m, vb[j * _K + r, sl])
                    smx[j, sl] = m
            pltpu.sync_copy(smx, mx.at[wid, blk])

        fire(0, vb0, sv0)

        def body(pair, carry):
            blk0 = 2 * pair
            blk1 = blk0 + 1
            fire(blk1, vb1, sv1)
            drain(blk0, vb0, sv0)
            compute(blk0, vb0)

            @pl.when(blk0 + 2 < _NBLK)
            def _():
                fire(blk0 + 2, vb0, sv0)

            drain(blk1, vb1, sv1)
            compute(blk1, vb1)
            return carry

        lax.fori_loop(0, _NBLK // 2, body, 0)


def _sc_max_gather(vflat, gidx1):
    f32 = jnp.float32
    kern = functools.partial(
        pl.kernel,
        mesh=plsc.VectorSubcoreMesh(core_axis_name="c", subcore_axis_name="s"),
        out_type=jax.ShapeDtypeStruct((_NWU, _NBLK, _BLKN, _C2P), f32),
        scratch_types=[
            pltpu.VMEM((_RPW * _K,), jnp.int32),
            pltpu.VMEM((_BLKN * _K, _C2P), f32),
            pltpu.VMEM((_BLKN * _K, _C2P), f32),
            pltpu.VMEM((_BLKN, _C2P), f32),
            pltpu.SemaphoreType.DMA,
            pltpu.SemaphoreType.DMA,
        ],
    )(_sc_body)
    return kern(vflat, gidx1)


def _phase3_body(u_ref, mx_ref, acc_ref, g2_ref, be2_ref, w2t_ref,
                 b2_ref, y3_ref, st3_ref):
    i = pl.program_id(0)
    ne = float(_B * _N * _K)
    mu = acc_ref[0:1, :] / ne
    var = acc_ref[1:2, :] / ne - mu * mu
    sc = g2_ref[...] / jnp.sqrt(var + _EPS)
    sh = be2_ref[...] - mu * sc
    upd = jnp.zeros((2, _C), jnp.float32)
    for s in range(_NB):
        z = u_ref[s] + mx_ref[s][:, :_C2]
        g = jnp.maximum(z * sc + sh, 0.0)
        y3 = jnp.dot(g, w2t_ref[...], preferred_element_type=jnp.float32)
        y3 = y3 + b2_ref[...]
        y3_ref[s] = y3
        sm = jnp.sum(y3, axis=0, keepdims=True)
        sq = jnp.sum(y3 * y3, axis=0, keepdims=True)
        upd = upd + jnp.concatenate([sm, sq], axis=0)

    @pl.when(i == 0)
    def _():
        st3_ref[...] = jnp.zeros_like(st3_ref)

    st3_ref[...] += upd


def _phase4_body(y3_ref, ep_ref, st3_ref, g3_ref, be3_ref, x_ref, o_ref):
    n3 = float(_B * _N)
    mu = st3_ref[0:1, :] / n3
    var = st3_ref[1:2, :] / n3 - mu * mu
    sc = g3_ref[...] / jnp.sqrt(var + _EPS)
    sh = be3_ref[...] - mu * sc
    for s in range(_NB):
        o = 0.8 * (y3_ref[s] * sc + sh) + 0.2 * ep_ref[s]
        oc = o[:_NPIX, :]
        o_ref[s] = oc.T + x_ref[s]


def kernel(x, fc1_W, fc1_b, fc1_g, fc1_be, nn_W, nn_b, nn_g, nn_be,
           fc2_W, fc2_b, fc2_g, fc2_be, node_prompts, graph_prompt,
           down_W, down_b, up_W, up_b):
    f32 = jnp.float32
    xf = x.reshape(_B, _C, _NPIX)
    at = (nn_W[:, :_C] - nn_W[:, _C:]).T
    bmt = nn_W[:, _C:].T
    dwt = down_W.T
    w2t = fc2_W.T
    uwt = up_W.T

    y1, st1 = pl.pallas_call(
        _phase1_body,
        grid=(_G,),
        in_specs=[
            pl.BlockSpec((_NB, _C, _NPIX), lambda b: (b, 0, 0)),
            pl.BlockSpec((_C, _C), lambda b: (0, 0)),
            pl.BlockSpec((_C, 1), lambda b: (0, 0)),
        ],
        out_specs=[
            pl.BlockSpec((_NB, _C, _NPIX), lambda b: (b, 0, 0)),
            pl.BlockSpec((_C, 2), lambda b: (0, 0)),
        ],
        out_shape=[
            jax.ShapeDtypeStruct((_B, _C, _NPIX), f32),
            jax.ShapeDtypeStruct((_C, 2), f32),
        ],
    )(xf, fc1_W, fc1_b.reshape(_C, 1))

    u, v, gidx, acc_e, ep = pl.pallas_call(
        _phase2_body,
        grid=(_G,),
        in_specs=[
            pl.BlockSpec((_NB, _C, _NPIX), lambda b: (b, 0, 0)),
            pl.BlockSpec((_C, 2), lambda b: (0, 0)),
            pl.BlockSpec((_C, 1), lambda b: (0, 0)),
            pl.BlockSpec((_C, 1), lambda b: (0, 0)),
            pl.BlockSpec((_C, _P), lambda b: (0, 0)),
            pl.BlockSpec((_C, _R), lambda b: (0, 0)),
            pl.BlockSpec((1, _R), lambda b: (0, 0)),
            pl.BlockSpec((_R, _C), lambda b: (0, 0)),
            pl.BlockSpec((_C, _C2), lambda b: (0, 0)),
            pl.BlockSpec((_C, _C2), lambda b: (0, 0)),
            pl.BlockSpec((1, _C2), lambda b: (0, 0)),
            pl.BlockSpec((_R, _C), lambda b: (0, 0)),
            pl.BlockSpec((1, _C), lambda b: (0, 0)),
        ],
        out_specs=[
            pl.BlockSpec((_NB, _N, _C2), lambda b: (b, 0, 0)),
            pl.BlockSpec((_NB, _N, _C2P), lambda b: (b, 0, 0)),
            pl.BlockSpec((_NB, _N, _K), lambda b: (b, 0, 0)),
            pl.BlockSpec((2, _C2), lambda b: (0, 0)),
            pl.BlockSpec((_NB, _N, _C), lambda b: (b, 0, 0)),
        ],
        out_shape=[
            jax.ShapeDtypeStruct((_B, _N, _C2), f32),
            jax.ShapeDtypeStruct((_B, _N, _C2P), f32),
            jax.ShapeDtypeStruct((_B, _N, _K), jnp.int32),
            jax.ShapeDtypeStruct((2, _C2), f32),
            jax.ShapeDtypeStruct((_B, _N, _C), f32),
        ],
    )(y1, st1, fc1_g.reshape(_C, 1), fc1_be.reshape(_C, 1), node_prompts,
      dwt, down_b.reshape(1, _R), graph_prompt, at, bmt,
      nn_b.reshape(1, _C2), uwt, up_b.reshape(1, _C))

    mx = _sc_max_gather(v.reshape(_ROWS, _C2P), gidx.reshape(_ROWS * _K))
    mx = mx.reshape(_B, _N, _C2P)

    y3, st3 = pl.pallas_call(
        _phase3_body,
        grid=(_G,),
        in_specs=[
            pl.BlockSpec((_NB, _N, _C2), lambda b: (b, 0, 0)),
            pl.BlockSpec((_NB, _N, _C2P), lambda b: (b, 0, 0)),
            pl.BlockSpec((2, _C2), lambda b: (0, 0)),
            pl.BlockSpec((1, _C2), lambda b: (0, 0)),
            pl.BlockSpec((1, _C2), lambda b: (0, 0)),
            pl.BlockSpec((_C2, _C), lambda b: (0, 0)),
            pl.BlockSpec((1, _C), lambda b: (0, 0)),
        ],
        out_specs=[
            pl.BlockSpec((_NB, _N, _C), lambda b: (b, 0, 0)),
            pl.BlockSpec((2, _C), lambda b: (0, 0)),
        ],
        out_shape=[
            jax.ShapeDtypeStruct((_B, _N, _C), f32),
            jax.ShapeDtypeStruct((2, _C), f32),
        ],
    )(u, mx, acc_e, nn_g.reshape(1, _C2), nn_be.reshape(1, _C2),
      w2t, fc2_b.reshape(1, _C))

    out = pl.pallas_call(
        _phase4_body,
        grid=(_G,),
        in_specs=[
            pl.BlockSpec((_NB, _N, _C), lambda b: (b, 0, 0)),
            pl.BlockSpec((_NB, _N, _C), lambda b: (b, 0, 0)),
            pl.BlockSpec((2, _C), lambda b: (0, 0)),
            pl.BlockSpec((1, _C), lambda b: (0, 0)),
            pl.BlockSpec((1, _C), lambda b: (0, 0)),
            pl.BlockSpec((_NB, _C, _NPIX), lambda b: (b, 0, 0)),
        ],
        out_specs=pl.BlockSpec((_NB, _C, _NPIX), lambda b: (b, 0, 0)),
        out_shape=jax.ShapeDtypeStruct((_B, _C, _NPIX), f32),
    )(y3, ep, st3, fc2_g.reshape(1, _C), fc2_be.reshape(1, _C), xf)

    return out.reshape(_B, _C, _H, _W)
